# initial kernel scaffold (unmeasured)
import jax
import jax.numpy as jnp
from jax import lax
from jax.experimental import pallas as pl
from jax.experimental.pallas import tpu as pltpu


def kernel(
    x,
):
    def body(*refs):
        pass

    out_shape = jax.ShapeDtypeStruct(..., jnp.float32)
    return pl.pallas_call(body, out_shape=out_shape)(...)


# baseline (device time: 80823 ns/iter reference)
import jax
import jax.numpy as jnp
from jax import lax
from jax.experimental import pallas as pl
from jax.experimental.pallas import tpu as pltpu

N_Z = 4


def kernel(x):
    m_per, n = x.shape
    half = m_per // 2

    def body(x_ref, out_ref, cw_ref, ccw_ref,
             cw_send_sems, cw_recv_sems, ccw_send_sems, ccw_recv_sems):
        my_x = lax.axis_index("x")
        my_y = lax.axis_index("y")
        my_z = lax.axis_index("z")
        right = (my_z + 1) % N_Z
        left = (my_z - 1) % N_Z

        barrier_sem = pltpu.get_barrier_semaphore()
        for nbr in (left, right):
            pl.semaphore_signal(
                barrier_sem, inc=1,
                device_id=(my_x, my_y, nbr),
                device_id_type=pl.DeviceIdType.MESH,
            )
        pl.semaphore_wait(barrier_sem, 2)

        out_ref[pl.ds(my_z * m_per, m_per), :] = x_ref[:, :]
        cw_ref[0, :, :] = x_ref[0:half, :]
        ccw_ref[0, :, :] = x_ref[half:m_per, :]

        for h in range(N_Z - 1):
            cw = pltpu.make_async_remote_copy(
                src_ref=cw_ref.at[h],
                dst_ref=cw_ref.at[h + 1],
                send_sem=cw_send_sems.at[h],
                recv_sem=cw_recv_sems.at[h],
                device_id=(my_x, my_y, right),
                device_id_type=pl.DeviceIdType.MESH,
            )
            ccw = pltpu.make_async_remote_copy(
                src_ref=ccw_ref.at[h],
                dst_ref=ccw_ref.at[h + 1],
                send_sem=ccw_send_sems.at[h],
                recv_sem=ccw_recv_sems.at[h],
                device_id=(my_x, my_y, left),
                device_id_type=pl.DeviceIdType.MESH,
            )
            cw.start()
            ccw.start()
            cw.wait()
            ccw.wait()

            origin_cw = (my_z - h - 1) % N_Z
            origin_ccw = (my_z + h + 1) % N_Z
            out_ref[pl.ds(origin_cw * m_per, half), :] = cw_ref[h + 1, :, :]
            out_ref[pl.ds(origin_ccw * m_per + half, half), :] = ccw_ref[h + 1, :, :]

    return pl.pallas_call(
        body,
        out_shape=jax.ShapeDtypeStruct((N_Z * m_per, n), x.dtype),
        in_specs=[pl.BlockSpec(memory_space=pltpu.VMEM)],
        out_specs=pl.BlockSpec(memory_space=pltpu.VMEM),
        scratch_shapes=[
            pltpu.VMEM((N_Z, half, n), x.dtype),
            pltpu.VMEM((N_Z, half, n), x.dtype),
            pltpu.SemaphoreType.DMA((N_Z - 1,)),
            pltpu.SemaphoreType.DMA((N_Z - 1,)),
            pltpu.SemaphoreType.DMA((N_Z - 1,)),
            pltpu.SemaphoreType.DMA((N_Z - 1,)),
        ],
        compiler_params=pltpu.CompilerParams(collective_id=0),
    )(x)


# device time: 65640 ns/iter; 1.2313x vs baseline; 1.2313x over previous
import jax
import jax.numpy as jnp
from jax import lax
from jax.experimental import pallas as pl
from jax.experimental.pallas import tpu as pltpu

N_Z = 4
N_RING = 8


def kernel(x):
    m_per, n = x.shape
    s_rows = m_per // N_RING

    def body(x_ref, out_ref, stage_ref, p1_ref, ring_ref,
             up_send, up_recv, dn_send, dn_recv,
             cw_send, cw_recv, ccw_send, ccw_recv, exit_sem):
        my_x = lax.axis_index("x")
        my_y = lax.axis_index("y")
        my_z = lax.axis_index("z")

        my_r = jnp.where(my_x == 0, my_y, 7 - my_y)

        def ring_coords(p):
            return jnp.where(p < 4, 0, 1), jnp.where(p < 4, p, 7 - p)

        rx, ry = ring_coords((my_r + 1) % N_RING)
        lx, ly = ring_coords((my_r - 1) % N_RING)
        tz_up = jnp.minimum(my_z + 1, N_Z - 1)
        tz_dn = jnp.maximum(my_z - 1, 0)

        partners = [
            (rx, ry, my_z),
            (lx, ly, my_z),
            (my_x, my_y, (my_z + 1) % N_Z),
            (my_x, my_y, (my_z - 1) % N_Z),
        ]

        barrier = pltpu.get_barrier_semaphore()
        for pid in partners:
            pl.semaphore_signal(
                barrier, inc=1, device_id=pid,
                device_id_type=pl.DeviceIdType.MESH,
            )
        pl.semaphore_wait(barrier, 4)

        out_ref[pl.ds(my_z * m_per, m_per), :] = x_ref[:, :]
        stage_ref[:, :] = x_ref[pl.ds(my_r * s_rows, s_rows), :]

        for h in range(N_Z - 1):
            up_tx = (my_z >= h) & (my_z < N_Z - 1)
            dn_tx = (my_z + h <= N_Z - 1) & (my_z > 0)
            up_rx = my_z >= h + 1
            dn_rx = my_z + h + 1 <= N_Z - 1

            @pl.when(up_tx)
            def _(h=h):
                src = stage_ref if h == 0 else p1_ref.at[my_z - h]
                pltpu.make_async_remote_copy(
                    src_ref=src, dst_ref=p1_ref.at[my_z - h],
                    send_sem=up_send.at[h], recv_sem=up_recv.at[h],
                    device_id=(my_x, my_y, tz_up),
                    device_id_type=pl.DeviceIdType.MESH,
                ).start()

            @pl.when(dn_tx)
            def _(h=h):
                src = stage_ref if h == 0 else p1_ref.at[my_z + h - 1]
                pltpu.make_async_remote_copy(
                    src_ref=src, dst_ref=p1_ref.at[my_z + h - 1],
                    send_sem=dn_send.at[h], recv_sem=dn_recv.at[h],
                    device_id=(my_x, my_y, tz_dn),
                    device_id_type=pl.DeviceIdType.MESH,
                ).start()

            @pl.when(up_rx)
            def _(h=h):
                pltpu.make_async_remote_copy(
                    src_ref=stage_ref, dst_ref=p1_ref.at[my_z - h - 1],
                    send_sem=up_send.at[h], recv_sem=up_recv.at[h],
                    device_id=(my_x, my_y, tz_dn),
                    device_id_type=pl.DeviceIdType.MESH,
                ).wait_recv()

            @pl.when(dn_rx)
            def _(h=h):
                pltpu.make_async_remote_copy(
                    src_ref=stage_ref, dst_ref=p1_ref.at[my_z + h],
                    send_sem=dn_send.at[h], recv_sem=dn_recv.at[h],
                    device_id=(my_x, my_y, tz_up),
                    device_id_type=pl.DeviceIdType.MESH,
                ).wait_recv()

            @pl.when(up_tx)
            def _(h=h):
                src = stage_ref if h == 0 else p1_ref.at[my_z - h]
                pltpu.make_async_remote_copy(
                    src_ref=src, dst_ref=p1_ref.at[my_z - h],
                    send_sem=up_send.at[h], recv_sem=up_recv.at[h],
                    device_id=(my_x, my_y, tz_up),
                    device_id_type=pl.DeviceIdType.MESH,
                ).wait_send()

            @pl.when(dn_tx)
            def _(h=h):
                src = stage_ref if h == 0 else p1_ref.at[my_z + h - 1]
                pltpu.make_async_remote_copy(
                    src_ref=src, dst_ref=p1_ref.at[my_z + h - 1],
                    send_sem=dn_send.at[h], recv_sem=dn_recv.at[h],
                    device_id=(my_x, my_y, tz_dn),
                    device_id_type=pl.DeviceIdType.MESH,
                ).wait_send()

        c = [jnp.where(i < my_z, i, i + 1) for i in range(3)]

        ring_ref[0, :, :, :] = p1_ref[:, :, :]
        for i in range(3):
            out_ref[pl.ds(c[i] * m_per + my_r * s_rows, s_rows), :] = p1_ref[i, :, :]

        for j in range(4):
            cwd = pltpu.make_async_remote_copy(
                src_ref=ring_ref.at[j], dst_ref=ring_ref.at[j + 1],
                send_sem=cw_send.at[j], recv_sem=cw_recv.at[j],
                device_id=(rx, ry, my_z),
                device_id_type=pl.DeviceIdType.MESH,
            )
            cwd.start()
            if j < 3:
                ccw_src = 0 if j == 0 else 8 - j
                ccwd = pltpu.make_async_remote_copy(
                    src_ref=ring_ref.at[ccw_src], dst_ref=ring_ref.at[7 - j],
                    send_sem=ccw_send.at[j], recv_sem=ccw_recv.at[j],
                    device_id=(lx, ly, my_z),
                    device_id_type=pl.DeviceIdType.MESH,
                )
                ccwd.start()
            cwd.wait()
            if j < 3:
                ccwd.wait()

            p_cw = (my_r - j - 1) % N_RING
            for i in range(3):
                out_ref[pl.ds(c[i] * m_per + p_cw * s_rows, s_rows), :] = \
                    ring_ref[j + 1, i, :, :]
            if j < 3:
                p_ccw = (my_r + j + 1) % N_RING
                for i in range(3):
                    out_ref[pl.ds(c[i] * m_per + p_ccw * s_rows, s_rows), :] = \
                        ring_ref[7 - j, i, :, :]

        for pid in partners:
            pl.semaphore_signal(
                exit_sem, inc=1, device_id=pid,
                device_id_type=pl.DeviceIdType.MESH,
            )
        pl.semaphore_wait(exit_sem, 4)

    return pl.pallas_call(
        body,
        out_shape=jax.ShapeDtypeStruct((N_Z * m_per, n), x.dtype),
        in_specs=[pl.BlockSpec(memory_space=pltpu.VMEM)],
        out_specs=pl.BlockSpec(memory_space=pltpu.VMEM),
        scratch_shapes=[
            pltpu.VMEM((s_rows, n), x.dtype),
            pltpu.VMEM((3, s_rows, n), x.dtype),
            pltpu.VMEM((N_RING, 3, s_rows, n), x.dtype),
            pltpu.SemaphoreType.DMA((N_Z - 1,)),
            pltpu.SemaphoreType.DMA((N_Z - 1,)),
            pltpu.SemaphoreType.DMA((N_Z - 1,)),
            pltpu.SemaphoreType.DMA((N_Z - 1,)),
            pltpu.SemaphoreType.DMA((4,)),
            pltpu.SemaphoreType.DMA((4,)),
            pltpu.SemaphoreType.DMA((3,)),
            pltpu.SemaphoreType.DMA((3,)),
            pltpu.SemaphoreType.REGULAR,
        ],
        compiler_params=pltpu.CompilerParams(collective_id=0),
    )(x)


# device time: 54532 ns/iter; 1.4821x vs baseline; 1.2037x over previous
import jax
import jax.numpy as jnp
from jax import lax
from jax.experimental import pallas as pl
from jax.experimental.pallas import tpu as pltpu

N_Z = 4
N_RING = 8


def kernel(x):
    m_per, n = x.shape
    s_rows = m_per // N_RING

    def body(x_ref, out_ref, stage_ref, ring_ref,
             up_send, up_recv, dn_send, dn_recv,
             cw_send, cw_recv, ccw_send, ccw_recv, exit_sem):
        my_x = lax.axis_index("x")
        my_y = lax.axis_index("y")
        my_z = lax.axis_index("z")

        my_r = jnp.where(my_x == 0, my_y, 7 - my_y)

        def ring_coords(p):
            return jnp.where(p < 4, 0, 1), jnp.where(p < 4, p, 7 - p)

        rx, ry = ring_coords((my_r + 1) % N_RING)
        lx, ly = ring_coords((my_r - 1) % N_RING)
        RIGHT = (rx, ry, my_z)
        LEFT = (lx, ly, my_z)
        UP = (my_x, my_y, jnp.minimum(my_z + 1, N_Z - 1))
        DN = (my_x, my_y, jnp.maximum(my_z - 1, 0))

        def rcopy(src, dst, ssem, rsem, dev):
            return pltpu.make_async_remote_copy(
                src_ref=src, dst_ref=dst, send_sem=ssem, recv_sem=rsem,
                device_id=dev, device_id_type=pl.DeviceIdType.MESH,
            )

        def slot(i, k):
            return i * N_RING + k

        cw_hops = [4, 3, 4]
        ccw_hops = [3, 4, 3]

        partners = [
            RIGHT, LEFT,
            (my_x, my_y, (my_z + 1) % N_Z),
            (my_x, my_y, (my_z - 1) % N_Z),
        ]
        barrier = pltpu.get_barrier_semaphore()
        for pid in partners:
            pl.semaphore_signal(
                barrier, inc=1, device_id=pid,
                device_id_type=pl.DeviceIdType.MESH,
            )
        pl.semaphore_wait(barrier, 4)

        stage_ref[:, :] = x_ref[pl.ds(my_r * s_rows, s_rows), :]

        @pl.when(my_z < N_Z - 1)
        def _():
            rcopy(stage_ref, ring_ref.at[my_z * N_RING],
                  up_send.at[0], up_recv.at[0], UP).start()

        @pl.when(my_z > 0)
        def _():
            rcopy(stage_ref, ring_ref.at[(my_z - 1) * N_RING],
                  dn_send.at[0], dn_recv.at[0], DN).start()

        out_ref[pl.ds(my_z * m_per, m_per), :] = x_ref[:, :]

        c = [jnp.where(i < my_z, i, i + 1) for i in range(3)]

        for h in range(N_Z - 1):
            up_rx = my_z >= h + 1
            dn_rx = my_z + h + 1 <= N_Z - 1

            @pl.when(up_rx)
            def _(h=h):
                iu = my_z - h - 1
                rcopy(stage_ref, ring_ref.at[iu * N_RING],
                      up_send.at[h], up_recv.at[h], DN).wait_recv()
                @pl.when((my_z >= h + 1) & (my_z < N_Z - 1))
                def _():
                    rcopy(ring_ref.at[iu * N_RING], ring_ref.at[iu * N_RING],
                          up_send.at[h + 1], up_recv.at[h + 1], UP).start()
                rcopy(ring_ref.at[iu * N_RING], ring_ref.at[iu * N_RING + 1],
                      cw_send.at[iu], cw_recv.at[iu * 4], RIGHT).start()
                rcopy(ring_ref.at[iu * N_RING], ring_ref.at[iu * N_RING + 7],
                      ccw_send.at[iu], ccw_recv.at[iu * 4], LEFT).start()

            @pl.when(dn_rx)
            def _(h=h):
                idn = my_z + h
                rcopy(stage_ref, ring_ref.at[idn * N_RING],
                      dn_send.at[h], dn_recv.at[h], UP).wait_recv()
                @pl.when((my_z + h + 1 <= N_Z - 1) & (my_z > 0))
                def _():
                    rcopy(ring_ref.at[idn * N_RING], ring_ref.at[idn * N_RING],
                          dn_send.at[h + 1], dn_recv.at[h + 1], DN).start()
                rcopy(ring_ref.at[idn * N_RING], ring_ref.at[idn * N_RING + 1],
                      cw_send.at[idn], cw_recv.at[idn * 4], RIGHT).start()
                rcopy(ring_ref.at[idn * N_RING], ring_ref.at[idn * N_RING + 7],
                      ccw_send.at[idn], ccw_recv.at[idn * 4], LEFT).start()

        for i in range(3):
            out_ref[pl.ds(c[i] * m_per + my_r * s_rows, s_rows), :] = \
                ring_ref[slot(i, 0), :, :]

        for j in range(4):
            for i in range(3):
                if j < cw_hops[i]:
                    rcopy(ring_ref.at[slot(i, j)], ring_ref.at[slot(i, j + 1)],
                          cw_send.at[i], cw_recv.at[i * 4 + j], RIGHT).wait_recv()
                    if j + 1 < cw_hops[i]:
                        rcopy(ring_ref.at[slot(i, j)], ring_ref.at[slot(i, j + 1)],
                              cw_send.at[i], cw_recv.at[i * 4 + j], RIGHT).wait_send()
                        rcopy(ring_ref.at[slot(i, j + 1)], ring_ref.at[slot(i, j + 2)],
                              cw_send.at[i], cw_recv.at[i * 4 + j + 1], RIGHT).start()
                if j < ccw_hops[i]:
                    ccw_src = slot(i, 0) if j == 0 else slot(i, 8 - j)
                    rcopy(ring_ref.at[ccw_src], ring_ref.at[slot(i, 7 - j)],
                          ccw_send.at[i], ccw_recv.at[i * 4 + j], LEFT).wait_recv()
                    if j + 1 < ccw_hops[i]:
                        rcopy(ring_ref.at[ccw_src], ring_ref.at[slot(i, 7 - j)],
                              ccw_send.at[i], ccw_recv.at[i * 4 + j], LEFT).wait_send()
                        rcopy(ring_ref.at[slot(i, 7 - j)], ring_ref.at[slot(i, 6 - j)],
                              ccw_send.at[i], ccw_recv.at[i * 4 + j + 1], LEFT).start()
            p_cw = (my_r - j - 1) % N_RING
            p_ccw = (my_r + j + 1) % N_RING
            for i in range(3):
                if j < cw_hops[i]:
                    out_ref[pl.ds(c[i] * m_per + p_cw * s_rows, s_rows), :] = \
                        ring_ref[slot(i, j + 1), :, :]
                if j < ccw_hops[i]:
                    out_ref[pl.ds(c[i] * m_per + p_ccw * s_rows, s_rows), :] = \
                        ring_ref[slot(i, 7 - j), :, :]

        for h in range(N_Z - 1):
            @pl.when((my_z >= h) & (my_z < N_Z - 1))
            def _(h=h):
                src = stage_ref if h == 0 else ring_ref.at[(my_z - h) * N_RING]
                rcopy(src, ring_ref.at[(my_z - h) * N_RING],
                      up_send.at[h], up_recv.at[h], UP).wait_send()

            @pl.when((my_z + h <= N_Z - 1) & (my_z > 0))
            def _(h=h):
                src = stage_ref if h == 0 else ring_ref.at[(my_z + h - 1) * N_RING]
                rcopy(src, ring_ref.at[(my_z + h - 1) * N_RING],
                      dn_send.at[h], dn_recv.at[h], DN).wait_send()

        for i in range(3):
            jl = cw_hops[i] - 1
            rcopy(ring_ref.at[slot(i, jl)], ring_ref.at[slot(i, jl + 1)],
                  cw_send.at[i], cw_recv.at[i * 4 + jl], RIGHT).wait_send()
            jl = ccw_hops[i] - 1
            src = slot(i, 0) if jl == 0 else slot(i, 8 - jl)
            rcopy(ring_ref.at[src], ring_ref.at[slot(i, 7 - jl)],
                  ccw_send.at[i], ccw_recv.at[i * 4 + jl], LEFT).wait_send()

        for pid in partners:
            pl.semaphore_signal(
                exit_sem, inc=1, device_id=pid,
                device_id_type=pl.DeviceIdType.MESH,
            )
        pl.semaphore_wait(exit_sem, 4)

    return pl.pallas_call(
        body,
        out_shape=jax.ShapeDtypeStruct((N_Z * m_per, n), x.dtype),
        in_specs=[pl.BlockSpec(memory_space=pltpu.VMEM)],
        out_specs=pl.BlockSpec(memory_space=pltpu.VMEM),
        scratch_shapes=[
            pltpu.VMEM((s_rows, n), x.dtype),
            pltpu.VMEM((3 * N_RING, s_rows, n), x.dtype),
            pltpu.SemaphoreType.DMA((N_Z - 1,)),
            pltpu.SemaphoreType.DMA((N_Z - 1,)),
            pltpu.SemaphoreType.DMA((N_Z - 1,)),
            pltpu.SemaphoreType.DMA((N_Z - 1,)),
            pltpu.SemaphoreType.DMA((3,)),
            pltpu.SemaphoreType.DMA((12,)),
            pltpu.SemaphoreType.DMA((3,)),
            pltpu.SemaphoreType.DMA((12,)),
            pltpu.SemaphoreType.REGULAR,
        ],
        compiler_params=pltpu.CompilerParams(collective_id=0),
    )(x)


# device time: 52225 ns/iter; 1.5476x vs baseline; 1.0442x over previous
import jax
import jax.numpy as jnp
from jax import lax
from jax.experimental import pallas as pl
from jax.experimental.pallas import tpu as pltpu

N_Z = 4
N_RING = 8


def kernel(x):
    m_per, n = x.shape
    s_rows = m_per // N_RING

    def body(x_ref, out_ref, stage_ref, ring_ref,
             z_send, z_recv, cw_send, cw_recv, ccw_send, ccw_recv, exit_sem):
        my_x = lax.axis_index("x")
        my_y = lax.axis_index("y")
        my_z = lax.axis_index("z")

        my_r = jnp.where(my_x == 0, my_y, 7 - my_y)

        def ring_coords(p):
            return jnp.where(p < 4, 0, 1), jnp.where(p < 4, p, 7 - p)

        rx, ry = ring_coords((my_r + 1) % N_RING)
        lx, ly = ring_coords((my_r - 1) % N_RING)
        RIGHT = (rx, ry, my_z)
        LEFT = (lx, ly, my_z)
        z_up = jnp.minimum(my_z + 1, N_Z - 1)
        z_dn = jnp.maximum(my_z - 1, 0)
        UP = (my_x, my_y, z_up)
        DN = (my_x, my_y, z_dn)

        def rcopy(src, dst, ssem, rsem, dev):
            return pltpu.make_async_remote_copy(
                src_ref=src, dst_ref=dst, send_sem=ssem, recv_sem=rsem,
                device_id=dev, device_id_type=pl.DeviceIdType.MESH,
            )

        def ordz(ch, zz):
            m = jnp.abs(ch - zz)
            o = (jnp.minimum(m - 1, zz) + jnp.minimum(m - 1, N_Z - 1 - zz)
                 + jnp.where((ch > zz) & (m <= zz), 1, 0))
            return jnp.clip(o, 0, 2)

        chunk_of = [
            jnp.where(my_z > 0, my_z - 1, 1),
            jnp.where(my_z <= 1, 2, jnp.where(my_z == 2, 3, 1)),
            jnp.where(my_z <= 1, 3, 0),
        ]

        def slot(d, k):
            return d * N_RING + k

        cw_hops = [4, 3, 4]
        ccw_hops = [3, 4, 3]

        partners = [
            RIGHT, LEFT,
            (my_x, my_y, (my_z + 1) % N_Z),
            (my_x, my_y, (my_z - 1) % N_Z),
        ]
        barrier = pltpu.get_barrier_semaphore()
        for pid in partners:
            pl.semaphore_signal(
                barrier, inc=1, device_id=pid,
                device_id_type=pl.DeviceIdType.MESH,
            )
        pl.semaphore_wait(barrier, 4)

        stage_ref[:, :] = x_ref[pl.ds(my_r * s_rows, s_rows), :]

        @pl.when(my_z < N_Z - 1)
        def _():
            rcopy(stage_ref, ring_ref.at[ordz(my_z, z_up) * N_RING],
                  z_send.at[0], z_recv.at[ordz(my_z, z_up)], UP).start()

        @pl.when(my_z > 0)
        def _():
            rcopy(stage_ref, ring_ref.at[ordz(my_z, z_dn) * N_RING],
                  z_send.at[1], z_recv.at[ordz(my_z, z_dn)], DN).start()

        out_ref[pl.ds(my_z * m_per, m_per), :] = x_ref[:, :]

        for d in range(3):
            rcopy(stage_ref, ring_ref.at[slot(d, 0)],
                  z_send.at[0], z_recv.at[d], UP).wait_recv()
            ch = chunk_of[d]

            @pl.when((ch < my_z) & (my_z < N_Z - 1))
            def _(d=d, ch=ch):
                rcopy(ring_ref.at[slot(d, 0)],
                      ring_ref.at[ordz(ch, z_up) * N_RING],
                      z_send.at[2 + d], z_recv.at[ordz(ch, z_up)], UP).start()

            @pl.when((ch > my_z) & (my_z > 0))
            def _(d=d, ch=ch):
                rcopy(ring_ref.at[slot(d, 0)],
                      ring_ref.at[ordz(ch, z_dn) * N_RING],
                      z_send.at[2 + d], z_recv.at[ordz(ch, z_dn)], DN).start()

            rcopy(ring_ref.at[slot(d, 0)], ring_ref.at[slot(d, 1)],
                  cw_send.at[d], cw_recv.at[d * 4], RIGHT).start()
            rcopy(ring_ref.at[slot(d, 0)], ring_ref.at[slot(d, 7)],
                  ccw_send.at[d], ccw_recv.at[d * 4], LEFT).start()

        for d in range(3):
            out_ref[pl.ds(chunk_of[d] * m_per + my_r * s_rows, s_rows), :] = \
                ring_ref[slot(d, 0), :, :]

        for j in range(4):
            for d in range(3):
                if j < cw_hops[d]:
                    rcopy(ring_ref.at[slot(d, j)], ring_ref.at[slot(d, j + 1)],
                          cw_send.at[d], cw_recv.at[d * 4 + j], RIGHT).wait_recv()
                    if j + 1 < cw_hops[d]:
                        rcopy(ring_ref.at[slot(d, j)], ring_ref.at[slot(d, j + 1)],
                              cw_send.at[d], cw_recv.at[d * 4 + j], RIGHT).wait_send()
                        rcopy(ring_ref.at[slot(d, j + 1)], ring_ref.at[slot(d, j + 2)],
                              cw_send.at[d], cw_recv.at[d * 4 + j + 1], RIGHT).start()
                if j < ccw_hops[d]:
                    ccw_src = slot(d, 0) if j == 0 else slot(d, 8 - j)
                    rcopy(ring_ref.at[ccw_src], ring_ref.at[slot(d, 7 - j)],
                          ccw_send.at[d], ccw_recv.at[d * 4 + j], LEFT).wait_recv()
                    if j + 1 < ccw_hops[d]:
                        rcopy(ring_ref.at[ccw_src], ring_ref.at[slot(d, 7 - j)],
                              ccw_send.at[d], ccw_recv.at[d * 4 + j], LEFT).wait_send()
                        rcopy(ring_ref.at[slot(d, 7 - j)], ring_ref.at[slot(d, 6 - j)],
                              ccw_send.at[d], ccw_recv.at[d * 4 + j + 1], LEFT).start()
            p_cw = (my_r - j - 1) % N_RING
            p_ccw = (my_r + j + 1) % N_RING
            for d in range(3):
                if j < cw_hops[d]:
                    out_ref[pl.ds(chunk_of[d] * m_per + p_cw * s_rows, s_rows), :] = \
                        ring_ref[slot(d, j + 1), :, :]
                if j < ccw_hops[d]:
                    out_ref[pl.ds(chunk_of[d] * m_per + p_ccw * s_rows, s_rows), :] = \
                        ring_ref[slot(d, 7 - j), :, :]

        @pl.when(my_z < N_Z - 1)
        def _():
            rcopy(stage_ref, ring_ref.at[0], z_send.at[0], z_recv.at[0],
                  UP).wait_send()

        @pl.when(my_z > 0)
        def _():
            rcopy(stage_ref, ring_ref.at[0], z_send.at[1], z_recv.at[0],
                  DN).wait_send()

        for d in range(3):
            ch = chunk_of[d]
            fwd = ((ch < my_z) & (my_z < N_Z - 1)) | ((ch > my_z) & (my_z > 0))

            @pl.when(fwd)
            def _(d=d):
                rcopy(ring_ref.at[slot(d, 0)], ring_ref.at[slot(d, 0)],
                      z_send.at[2 + d], z_recv.at[d], UP).wait_send()

            jl = cw_hops[d] - 1
            rcopy(ring_ref.at[slot(d, jl)], ring_ref.at[slot(d, jl + 1)],
                  cw_send.at[d], cw_recv.at[d * 4 + jl], RIGHT).wait_send()
            jl = ccw_hops[d] - 1
            src = slot(d, 0) if jl == 0 else slot(d, 8 - jl)
            rcopy(ring_ref.at[src], ring_ref.at[slot(d, 7 - jl)],
                  ccw_send.at[d], ccw_recv.at[d * 4 + jl], LEFT).wait_send()

        for pid in partners:
            pl.semaphore_signal(
                exit_sem, inc=1, device_id=pid,
                device_id_type=pl.DeviceIdType.MESH,
            )
        pl.semaphore_wait(exit_sem, 4)

    return pl.pallas_call(
        body,
        out_shape=jax.ShapeDtypeStruct((N_Z * m_per, n), x.dtype),
        in_specs=[pl.BlockSpec(memory_space=pltpu.VMEM)],
        out_specs=pl.BlockSpec(memory_space=pltpu.VMEM),
        scratch_shapes=[
            pltpu.VMEM((s_rows, n), x.dtype),
            pltpu.VMEM((3 * N_RING, s_rows, n), x.dtype),
            pltpu.SemaphoreType.DMA((5,)),
            pltpu.SemaphoreType.DMA((3,)),
            pltpu.SemaphoreType.DMA((3,)),
            pltpu.SemaphoreType.DMA((12,)),
            pltpu.SemaphoreType.DMA((3,)),
            pltpu.SemaphoreType.DMA((12,)),
            pltpu.SemaphoreType.REGULAR,
        ],
        compiler_params=pltpu.CompilerParams(collective_id=0),
    )(x)


# device time: 52164 ns/iter; 1.5494x vs baseline; 1.0012x over previous
import jax
import jax.numpy as jnp
from jax import lax
from jax.experimental import pallas as pl
from jax.experimental.pallas import tpu as pltpu

N_Z = 4
N_RING = 8


def kernel(x):
    m_per, n = x.shape
    s_rows = m_per // N_RING

    def body(x_ref, out_ref, stage_ref,
             z_send, z_recv, cw_send, cw_recv, ccw_send, ccw_recv, exit_sem):
        my_x = lax.axis_index("x")
        my_y = lax.axis_index("y")
        my_z = lax.axis_index("z")

        my_r = jnp.where(my_x == 0, my_y, 7 - my_y)

        def ring_coords(p):
            return jnp.where(p < 4, 0, 1), jnp.where(p < 4, p, 7 - p)

        rx, ry = ring_coords((my_r + 1) % N_RING)
        lx, ly = ring_coords((my_r - 1) % N_RING)
        RIGHT = (rx, ry, my_z)
        LEFT = (lx, ly, my_z)
        z_up = jnp.minimum(my_z + 1, N_Z - 1)
        z_dn = jnp.maximum(my_z - 1, 0)
        UP = (my_x, my_y, z_up)
        DN = (my_x, my_y, z_dn)

        def rcopy(src, dst, ssem, rsem, dev):
            return pltpu.make_async_remote_copy(
                src_ref=src, dst_ref=dst, send_sem=ssem, recv_sem=rsem,
                device_id=dev, device_id_type=pl.DeviceIdType.MESH,
            )

        def rowref(ch, origin):
            return out_ref.at[pl.ds(ch * m_per + origin * s_rows, s_rows), :]

        def ordz(ch, zz):
            m = jnp.abs(ch - zz)
            o = (jnp.minimum(m - 1, zz) + jnp.minimum(m - 1, N_Z - 1 - zz)
                 + jnp.where((ch > zz) & (m <= zz), 1, 0))
            return jnp.clip(o, 0, 2)

        chunk_of = [
            jnp.where(my_z > 0, my_z - 1, 1),
            jnp.where(my_z <= 1, 2, jnp.where(my_z == 2, 3, 1)),
            jnp.where(my_z <= 1, 3, 0),
        ]

        cw_hops = [4, 3, 4]
        ccw_hops = [3, 4, 3]

        partners = [
            RIGHT, LEFT,
            (my_x, my_y, (my_z + 1) % N_Z),
            (my_x, my_y, (my_z - 1) % N_Z),
        ]
        barrier = pltpu.get_barrier_semaphore()
        for pid in partners:
            pl.semaphore_signal(
                barrier, inc=1, device_id=pid,
                device_id_type=pl.DeviceIdType.MESH,
            )
        pl.semaphore_wait(barrier, 4)

        stage_ref[:, :] = x_ref[pl.ds(my_r * s_rows, s_rows), :]

        @pl.when(my_z < N_Z - 1)
        def _():
            rcopy(stage_ref, rowref(my_z, my_r),
                  z_send.at[0], z_recv.at[ordz(my_z, z_up)], UP).start()

        @pl.when(my_z > 0)
        def _():
            rcopy(stage_ref, rowref(my_z, my_r),
                  z_send.at[1], z_recv.at[ordz(my_z, z_dn)], DN).start()

        out_ref[pl.ds(my_z * m_per, m_per), :] = x_ref[:, :]

        for d in range(3):
            ch = chunk_of[d]
            rcopy(stage_ref, rowref(ch, my_r),
                  z_send.at[0], z_recv.at[d], UP).wait_recv()

            @pl.when((ch < my_z) & (my_z < N_Z - 1))
            def _(d=d, ch=ch):
                rcopy(rowref(ch, my_r), rowref(ch, my_r),
                      z_send.at[2 + d], z_recv.at[ordz(ch, z_up)], UP).start()

            @pl.when((ch > my_z) & (my_z > 0))
            def _(d=d, ch=ch):
                rcopy(rowref(ch, my_r), rowref(ch, my_r),
                      z_send.at[2 + d], z_recv.at[ordz(ch, z_dn)], DN).start()

            rcopy(rowref(ch, my_r), rowref(ch, my_r),
                  cw_send.at[d], cw_recv.at[d * 4], RIGHT).start()
            rcopy(rowref(ch, my_r), rowref(ch, my_r),
                  ccw_send.at[d], ccw_recv.at[d * 4], LEFT).start()

        for j in range(4):
            for d in range(3):
                ch = chunk_of[d]
                if j < cw_hops[d]:
                    o_in = (my_r - j - 1) % N_RING
                    rcopy(rowref(ch, o_in), rowref(ch, o_in),
                          cw_send.at[d], cw_recv.at[d * 4 + j], RIGHT).wait_recv()
                    if j + 1 < cw_hops[d]:
                        rcopy(rowref(ch, o_in), rowref(ch, o_in),
                              cw_send.at[d], cw_recv.at[d * 4 + j], RIGHT).wait_send()
                        rcopy(rowref(ch, o_in), rowref(ch, o_in),
                              cw_send.at[d], cw_recv.at[d * 4 + j + 1], RIGHT).start()
                if j < ccw_hops[d]:
                    o_in = (my_r + j + 1) % N_RING
                    rcopy(rowref(ch, o_in), rowref(ch, o_in),
                          ccw_send.at[d], ccw_recv.at[d * 4 + j], LEFT).wait_recv()
                    if j + 1 < ccw_hops[d]:
                        rcopy(rowref(ch, o_in), rowref(ch, o_in),
                              ccw_send.at[d], ccw_recv.at[d * 4 + j], LEFT).wait_send()
                        rcopy(rowref(ch, o_in), rowref(ch, o_in),
                              ccw_send.at[d], ccw_recv.at[d * 4 + j + 1], LEFT).start()

        @pl.when(my_z < N_Z - 1)
        def _():
            rcopy(stage_ref, rowref(my_z, my_r), z_send.at[0], z_recv.at[0],
                  UP).wait_send()

        @pl.when(my_z > 0)
        def _():
            rcopy(stage_ref, rowref(my_z, my_r), z_send.at[1], z_recv.at[0],
                  DN).wait_send()

        for d in range(3):
            ch = chunk_of[d]
            fwd = ((ch < my_z) & (my_z < N_Z - 1)) | ((ch > my_z) & (my_z > 0))

            @pl.when(fwd)
            def _(d=d, ch=ch):
                rcopy(rowref(ch, my_r), rowref(ch, my_r),
                      z_send.at[2 + d], z_recv.at[d], UP).wait_send()

            jl = cw_hops[d] - 1
            o_in = (my_r - jl - 1) % N_RING
            rcopy(rowref(ch, o_in), rowref(ch, o_in),
                  cw_send.at[d], cw_recv.at[d * 4 + jl], RIGHT).wait_send()
            jl = ccw_hops[d] - 1
            o_in = (my_r + jl + 1) % N_RING
            rcopy(rowref(ch, o_in), rowref(ch, o_in),
                  ccw_send.at[d], ccw_recv.at[d * 4 + jl], LEFT).wait_send()

        for pid in partners:
            pl.semaphore_signal(
                exit_sem, inc=1, device_id=pid,
                device_id_type=pl.DeviceIdType.MESH,
            )
        pl.semaphore_wait(exit_sem, 4)

    return pl.pallas_call(
        body,
        out_shape=jax.ShapeDtypeStruct((N_Z * m_per, n), x.dtype),
        in_specs=[pl.BlockSpec(memory_space=pltpu.VMEM)],
        out_specs=pl.BlockSpec(memory_space=pltpu.VMEM),
        scratch_shapes=[
            pltpu.VMEM((s_rows, n), x.dtype),
            pltpu.SemaphoreType.DMA((5,)),
            pltpu.SemaphoreType.DMA((3,)),
            pltpu.SemaphoreType.DMA((3,)),
            pltpu.SemaphoreType.DMA((12,)),
            pltpu.SemaphoreType.DMA((3,)),
            pltpu.SemaphoreType.DMA((12,)),
            pltpu.SemaphoreType.REGULAR,
        ],
        compiler_params=pltpu.CompilerParams(collective_id=0),
    )(x)
